# baseline (speedup 1.0000x reference)
import functools

import jax
import jax.numpy as jnp
import numpy as np
from jax import lax
from jax.experimental import pallas as pl
from jax.experimental.pallas import tpu as pltpu
from jax.experimental.pallas import tpu_sc as plsc

BB, CC, NN, KK = 8, 256, 2048, 16
CO = 256
TN = 256
NW = 32
CHP = 8
_RPC = CHP * KK
SPLITS = (4, 2, 2)
_MTOT = float(BB * NN * KK)

_base16 = np.arange(16)
_blk32 = np.stack([_base16, _base16 + 16], axis=1).reshape(32)
_PPERM = np.concatenate([32 * b + _blk32 for b in range(CO // 32)])


def _prep_body(x_ref, wa_ref, wd_ref, pt_ref, qt_ref, sq_ref):
    xb = x_ref[0]
    ptv = lax.dot_general(xb, wa_ref[...], (((0,), (1,)), ((), ())),
                          preferred_element_type=jnp.float32)
    pt_ref[0] = ptv.astype(jnp.bfloat16)
    qt_ref[0] = lax.dot_general(xb, wd_ref[...], (((0,), (1,)), ((), ())),
                                preferred_element_type=jnp.float32)
    sq_ref[0, 0] = jnp.sum(xb * xb, axis=0)


def _prep(x, wa, wd):
    return pl.pallas_call(
        _prep_body,
        grid=(BB,),
        in_specs=[
            pl.BlockSpec((1, CC, NN), lambda b: (b, 0, 0)),
            pl.BlockSpec((CO, CC), lambda b: (0, 0)),
            pl.BlockSpec((CO, CC), lambda b: (0, 0)),
        ],
        out_specs=[
            pl.BlockSpec((1, NN, CO), lambda b: (b, 0, 0)),
            pl.BlockSpec((1, NN, CO), lambda b: (b, 0, 0)),
            pl.BlockSpec((1, 1, NN), lambda b: (b, 0, 0)),
        ],
        out_shape=[
            jax.ShapeDtypeStruct((BB, NN, CO), jnp.bfloat16),
            jax.ShapeDtypeStruct((BB, NN, CO), jnp.float32),
            jax.ShapeDtypeStruct((BB, 1, NN), jnp.float32),
        ],
    )(x, wa, wd)


def _knn_body(off, xrow_ref, xall_ref, sq_ref, sqrow_ref, idx_ref):
    b = pl.program_id(0)
    xr = xrow_ref[0]
    xa = xall_ref[0]
    inner = lax.dot_general(xr, xa, (((0,), (0,)), ((), ())),
                            preferred_element_type=jnp.float32)
    sq = sq_ref[0]
    sqr = sqrow_ref[0]
    d = jnp.transpose(sqr) + sq - 2.0 * inner
    col_ids = lax.broadcasted_iota(jnp.int32, (TN, NN), 1)
    cols = []
    for _ in range(KK):
        am = jnp.argmin(d, axis=1).astype(jnp.int32).reshape(TN, 1)
        cols.append(am)
        d = jnp.where(col_ids == am, jnp.inf, d)
    idx_ref[0] = jnp.concatenate(cols, axis=1) + (b + off) * NN


def _knn(x, sq, off, nb):
    return pl.pallas_call(
        functools.partial(_knn_body, off),
        grid=(nb, NN // TN),
        in_specs=[
            pl.BlockSpec((1, CC, TN), lambda b, j: (b + off, 0, j)),
            pl.BlockSpec((1, CC, NN), lambda b, j: (b + off, 0, 0)),
            pl.BlockSpec((1, 1, NN), lambda b, j: (b + off, 0, 0)),
            pl.BlockSpec((1, 1, TN), lambda b, j: (b + off, 0, j)),
        ],
        out_specs=pl.BlockSpec((1, TN, KK), lambda b, j: (b, j, 0)),
        out_shape=jax.ShapeDtypeStruct((nb, NN, KK), jnp.int32),
    )(x, x, sq, sq)


def _sc_gather_reduce(pt_flat, idx2d, nb):
    ppw = nb * NN // NW
    nch = ppw // CHP
    mesh = plsc.VectorSubcoreMesh(core_axis_name="c", subcore_axis_name="s")

    @functools.partial(
        pl.kernel,
        mesh=mesh,
        out_type=(
            jax.ShapeDtypeStruct((nb * NN, CO), jnp.float32),
            jax.ShapeDtypeStruct((nb * NN, CO), jnp.float32),
            jax.ShapeDtypeStruct((nb * NN, CO), jnp.float32),
        ),
        name="sc_gather_reduce",
        scratch_types=[
            pltpu.VMEM((nch, _RPC), jnp.int32),
            pltpu.VMEM((_RPC, CO // 2), jnp.int32),
            pltpu.VMEM((_RPC, CO // 2), jnp.int32),
            pltpu.VMEM((CHP, CO), jnp.float32),
            pltpu.VMEM((CHP, CO), jnp.float32),
            pltpu.VMEM((CHP, CO), jnp.float32),
            pltpu.VMEM((CHP, CO), jnp.float32),
            pltpu.VMEM((CHP, CO), jnp.float32),
            pltpu.VMEM((CHP, CO), jnp.float32),
            pltpu.SemaphoreType.DMA,
            pltpu.SemaphoreType.DMA,
            pltpu.SemaphoreType.DMA,
            pltpu.SemaphoreType.DMA,
        ],
    )
    def body(pt_hbm, idx_hbm, mx_hbm, s1_hbm, s2_hbm,
             idx_v, rows_v0, rows_v1, mx_v0, mx_v1, s1_v0, s1_v1,
             s2_v0, s2_v1, gsem0, gsem1, osem0, osem1):
        wid = lax.axis_index("s") * 2 + lax.axis_index("c")
        base_pt = wid * ppw
        rows_b = (rows_v0, rows_v1)
        mx_b = (mx_v0, mx_v1)
        s1_b = (s1_v0, s1_v1)
        s2_b = (s2_v0, s2_v1)
        gsem = (gsem0, gsem1)
        osem = (osem0, osem1)

        pltpu.sync_copy(idx_hbm.at[pl.ds(wid * nch, nch)], idx_v)

        def start_gather(ci, par):
            pltpu.make_async_copy(
                pt_hbm.at[idx_v.at[ci]], rows_b[par], gsem[par]).start()

        def wait_gather(ci, par):
            pltpu.make_async_copy(
                pt_hbm.at[idx_v.at[ci]], rows_b[par], gsem[par]).wait()

        def compute_chunk(ci, par):
            rows_v = rows_b[par]
            mx_v, s1_v, s2_v = mx_b[par], s1_b[par], s2_b[par]

            def point_body(p, _):
                base = p * KK
                himask = jnp.int32(-65536)

                def unpk(row, ws):
                    w = rows_v[row, pl.ds(ws, 16)]
                    a = lax.bitcast_convert_type(lax.shift_left(w, 16),
                                                 jnp.float32)
                    b = lax.bitcast_convert_type(lax.bitwise_and(w, himask),
                                                 jnp.float32)
                    return a, b

                def col_body(wb, _):
                    ws = wb * 16
                    nat = wb * 32
                    va, vb = unpk(base, ws)
                    ma, mb = va, vb
                    sa, sb = va, vb
                    qa, qb = va * va, vb * vb
                    for r in range(1, KK):
                        va, vb = unpk(base + r, ws)
                        ma = jnp.maximum(ma, va)
                        mb = jnp.maximum(mb, vb)
                        sa = sa + va
                        sb = sb + vb
                        qa = qa + va * va
                        qb = qb + vb * vb
                    mx_v[p, pl.ds(nat, 16)] = ma
                    mx_v[p, pl.ds(nat + 16, 16)] = mb
                    s1_v[p, pl.ds(nat, 16)] = sa
                    s1_v[p, pl.ds(nat + 16, 16)] = sb
                    s2_v[p, pl.ds(nat, 16)] = qa
                    s2_v[p, pl.ds(nat + 16, 16)] = qb
                    return 0

                lax.fori_loop(0, CO // 32, col_body, 0)
                return 0

            lax.fori_loop(0, CHP, point_body, 0)
            p0 = base_pt + ci * CHP
            pltpu.make_async_copy(mx_v, mx_hbm.at[pl.ds(p0, CHP)],
                                  osem[par]).start()
            pltpu.make_async_copy(s1_v, s1_hbm.at[pl.ds(p0, CHP)],
                                  osem[par]).start()
            pltpu.make_async_copy(s2_v, s2_hbm.at[pl.ds(p0, CHP)],
                                  osem[par]).start()

        def wait_out(par):
            pltpu.make_async_copy(mx_b[par], mx_hbm.at[pl.ds(0, CHP)],
                                  osem[par]).wait()
            pltpu.make_async_copy(s1_b[par], s1_hbm.at[pl.ds(0, CHP)],
                                  osem[par]).wait()
            pltpu.make_async_copy(s2_b[par], s2_hbm.at[pl.ds(0, CHP)],
                                  osem[par]).wait()

        start_gather(0, 0)

        def pair_body(pi, _):
            for q in range(2):
                ci = pi * 2 + q
                par = q

                @pl.when(ci + 1 < nch)
                def _qnext():
                    start_gather(ci + 1, 1 - par)

                wait_gather(ci, par)

                @pl.when(ci >= 2)
                def _drain():
                    wait_out(par)

                compute_chunk(ci, par)
            return 0

        lax.fori_loop(0, nch // 2, pair_body, 0)
        wait_out(0)
        wait_out(1)

    return body(pt_flat, idx2d)


_RED_T = 512


def _stats_body(nsteps, s1_ref, s2_ref, qt_ref, out_ref, acc_ref):
    step = pl.program_id(0)

    @pl.when(step == 0)
    def _init():
        acc_ref[...] = jnp.zeros_like(acc_ref)

    s1 = s1_ref[...]
    s2 = s2_ref[...]
    qt = qt_ref[0]
    sum_y = jnp.sum(s1 + KK * qt, axis=0, keepdims=True)
    sum_y2 = jnp.sum(s2 + 2.0 * s1 * qt + KK * qt * qt, axis=0,
                     keepdims=True)
    acc_ref[0:1, :] += sum_y
    acc_ref[1:2, :] += sum_y2

    @pl.when(step == nsteps - 1)
    def _fin():
        out_ref[...] = acc_ref[...]


def _stats(s1, s2, qt3, off, nb):
    nsteps = nb * NN // _RED_T
    soff = off * NN // _RED_T
    return pl.pallas_call(
        functools.partial(_stats_body, nsteps),
        grid=(nsteps,),
        in_specs=[
            pl.BlockSpec((_RED_T, CO), lambda i: (i, 0)),
            pl.BlockSpec((_RED_T, CO), lambda i: (i, 0)),
            pl.BlockSpec((1, _RED_T, CO), lambda i: (i + soff, 0, 0)),
        ],
        out_specs=pl.BlockSpec((2, CO), lambda i: (0, 0)),
        out_shape=jax.ShapeDtypeStruct((2, CO), jnp.float32),
        scratch_shapes=[pltpu.VMEM((2, CO), jnp.float32)],
    )(s1, s2, qt3)


def _fin_body(nsums, m_ref, qt_ref, g_ref, be_ref, w1_ref, b1_ref,
              w2_ref, b2_ref, *rest):
    sum_refs = rest[:nsums]
    out_ref = rest[nsums]
    acc = sum_refs[0][...]
    for r in sum_refs[1:]:
        acc = acc + r[...]
    mean = acc[0:1, :] / _MTOT
    var = acc[1:2, :] / _MTOT - mean * mean
    istd = lax.rsqrt(var + 1e-5)
    m = m_ref[0]
    qt = qt_ref[0]
    y = ((m + qt) - mean) * istd * g_ref[...] + be_ref[...]
    y = jnp.maximum(y, 0.0)
    w = jnp.sum(y, axis=0, keepdims=True) * (1.0 / NN)
    h = lax.dot_general(w, w1_ref[...], (((1,), (1,)), ((), ())),
                        preferred_element_type=jnp.float32)
    h = jnp.maximum(h + b1_ref[...], 0.0)
    s = lax.dot_general(h, w2_ref[...], (((1,), (1,)), ((), ())),
                        preferred_element_type=jnp.float32)
    s = jax.nn.sigmoid(s + b2_ref[...])
    out_ref[0] = jnp.transpose(y) * jnp.transpose(s)


def _finalize(mx, qt, sums, gamma2, beta2, W1, b12, W2, b22, off, nb):
    hid = W1.shape[0]
    nsums = len(sums)
    full = lambda b: (0, 0)
    return pl.pallas_call(
        functools.partial(_fin_body, nsums),
        grid=(nb,),
        in_specs=[
            pl.BlockSpec((1, NN, CO), lambda b: (b, 0, 0)),
            pl.BlockSpec((1, NN, CO), lambda b: (b + off, 0, 0)),
            pl.BlockSpec((1, CO), full),
            pl.BlockSpec((1, CO), full),
            pl.BlockSpec((hid, CO), full),
            pl.BlockSpec((1, hid), full),
            pl.BlockSpec((CO, hid), full),
            pl.BlockSpec((1, CO), full),
        ] + [pl.BlockSpec((2, CO), full) for _ in range(nsums)],
        out_specs=pl.BlockSpec((1, CO, NN), lambda b: (b, 0, 0)),
        out_shape=jax.ShapeDtypeStruct((nb, CO, NN), jnp.float32),
    )(mx, qt, gamma2, beta2, W1, b12, W2, b22, *sums)


def kernel(x, W_conv, gamma, beta, W1, b1, W2, b2):
    wa = W_conv[:, :CC]
    wd = W_conv[:, CC:] - wa
    wa_perm = wa[_PPERM]

    pt, qt, sq = _prep(x, wa_perm, wd)
    pt_flat = lax.bitcast_convert_type(
        pt.reshape(BB * NN, CO // 2, 2), jnp.int32)
    qt3 = qt.reshape(BB * NN // _RED_T, _RED_T, CO)

    mxs, sums = [], []
    off = 0
    for nb in SPLITS:
        idx = _knn(x, sq, off, nb)
        mx, s1, s2 = _sc_gather_reduce(
            pt_flat, idx.reshape(nb * NN * KK // _RPC, _RPC), nb)
        sums.append(_stats(s1, s2, qt3, off, nb))
        mxs.append(mx)
        off += nb

    outs = []
    off = 0
    for mx, nb in zip(mxs, SPLITS):
        outs.append(_finalize(
            mx.reshape(nb, NN, CO), qt, sums,
            gamma.reshape(1, CO), beta.reshape(1, CO),
            W1, b1.reshape(1, -1), W2, b2.reshape(1, CO), off, nb))
        off += nb
    return jnp.concatenate(outs, axis=0)


# speedup vs baseline: 1.1512x; 1.1512x over previous
import functools

import jax
import jax.numpy as jnp
import numpy as np
from jax import lax
from jax.experimental import pallas as pl
from jax.experimental.pallas import tpu as pltpu
from jax.experimental.pallas import tpu_sc as plsc

BB, CC, NN, KK = 8, 256, 2048, 16
CO = 256
TN = 256
NW = 32
CHP = 8
_RPC = CHP * KK
SPLITS = (2, 2, 2, 2)
_MTOT = float(BB * NN * KK)

_j = np.arange(128)
_PPERM = np.concatenate([32 * (_j // 16) + (_j % 16),
                         32 * (_j // 16) + 16 + (_j % 16)])


def _prep_body(x_ref, wa_ref, wd_ref, pt_ref, qt_ref, sq_ref):
    xb = x_ref[0]
    ptv = lax.dot_general(xb, wa_ref[...], (((0,), (1,)), ((), ())),
                          preferred_element_type=jnp.float32)

    def rnd(v):
        b = lax.bitcast_convert_type(v, jnp.int32)
        b = b + jnp.int32(0x7FFF) + lax.bitwise_and(
            lax.shift_right_logical(b, 16), jnp.int32(1))
        return lax.shift_right_arithmetic(b, 16)

    lo = lax.bitwise_and(rnd(ptv[:, :CO // 2]), jnp.int32(0xFFFF))
    hi = lax.shift_left(rnd(ptv[:, CO // 2:]), 16)
    pt_ref[0] = lax.bitwise_or(lo, hi)
    qt_ref[0] = lax.dot_general(xb, wd_ref[...], (((0,), (1,)), ((), ())),
                                preferred_element_type=jnp.float32)
    sq_ref[0, 0] = jnp.sum(xb * xb, axis=0)


def _prep(x, wa, wd):
    return pl.pallas_call(
        _prep_body,
        grid=(BB,),
        in_specs=[
            pl.BlockSpec((1, CC, NN), lambda b: (b, 0, 0)),
            pl.BlockSpec((CO, CC), lambda b: (0, 0)),
            pl.BlockSpec((CO, CC), lambda b: (0, 0)),
        ],
        out_specs=[
            pl.BlockSpec((1, NN, CO // 2), lambda b: (b, 0, 0)),
            pl.BlockSpec((1, NN, CO), lambda b: (b, 0, 0)),
            pl.BlockSpec((1, 1, NN), lambda b: (b, 0, 0)),
        ],
        out_shape=[
            jax.ShapeDtypeStruct((BB, NN, CO // 2), jnp.int32),
            jax.ShapeDtypeStruct((BB, NN, CO), jnp.float32),
            jax.ShapeDtypeStruct((BB, 1, NN), jnp.float32),
        ],
    )(x, wa, wd)


def _knn_body(off, xrow_ref, xall_ref, sq_ref, sqrow_ref, idx_ref):
    b = pl.program_id(0)
    xr = xrow_ref[0]
    xa = xall_ref[0]
    inner = lax.dot_general(xr, xa, (((0,), (0,)), ((), ())),
                            preferred_element_type=jnp.float32)
    sq = sq_ref[0]
    sqr = sqrow_ref[0]
    d = jnp.transpose(sqr) + sq - 2.0 * inner
    col_ids = lax.broadcasted_iota(jnp.int32, (TN, NN), 1)
    cols = []
    for _ in range(KK):
        am = jnp.argmin(d, axis=1).astype(jnp.int32).reshape(TN, 1)
        cols.append(am)
        d = jnp.where(col_ids == am, jnp.inf, d)
    idx_ref[0] = jnp.concatenate(cols, axis=1) + (b + off) * NN


def _knn(x, sq, off, nb):
    return pl.pallas_call(
        functools.partial(_knn_body, off),
        grid=(nb, NN // TN),
        in_specs=[
            pl.BlockSpec((1, CC, TN), lambda b, j: (b + off, 0, j)),
            pl.BlockSpec((1, CC, NN), lambda b, j: (b + off, 0, 0)),
            pl.BlockSpec((1, 1, NN), lambda b, j: (b + off, 0, 0)),
            pl.BlockSpec((1, 1, TN), lambda b, j: (b + off, 0, j)),
        ],
        out_specs=pl.BlockSpec((1, TN, KK), lambda b, j: (b, j, 0)),
        out_shape=jax.ShapeDtypeStruct((nb, NN, KK), jnp.int32),
    )(x, x, sq, sq)


def _sc_gather_reduce(pt_flat, idx2d, nb):
    ppw = nb * NN // NW
    nch = ppw // CHP
    mesh = plsc.VectorSubcoreMesh(core_axis_name="c", subcore_axis_name="s")

    @functools.partial(
        pl.kernel,
        mesh=mesh,
        out_type=(
            jax.ShapeDtypeStruct((nb * NN, CO), jnp.float32),
            jax.ShapeDtypeStruct((nb * NN, CO), jnp.float32),
            jax.ShapeDtypeStruct((nb * NN, CO), jnp.float32),
        ),
        name="sc_gather_reduce",
        scratch_types=[
            pltpu.VMEM((nch, _RPC), jnp.int32),
            pltpu.VMEM((_RPC, CO // 2), jnp.int32),
            pltpu.VMEM((_RPC, CO // 2), jnp.int32),
            pltpu.VMEM((CHP, CO), jnp.float32),
            pltpu.VMEM((CHP, CO), jnp.float32),
            pltpu.VMEM((CHP, CO), jnp.float32),
            pltpu.VMEM((CHP, CO), jnp.float32),
            pltpu.VMEM((CHP, CO), jnp.float32),
            pltpu.VMEM((CHP, CO), jnp.float32),
            pltpu.SemaphoreType.DMA,
            pltpu.SemaphoreType.DMA,
            pltpu.SemaphoreType.DMA,
            pltpu.SemaphoreType.DMA,
        ],
    )
    def body(pt_hbm, idx_hbm, mx_hbm, s1_hbm, s2_hbm,
             idx_v, rows_v0, rows_v1, mx_v0, mx_v1, s1_v0, s1_v1,
             s2_v0, s2_v1, gsem0, gsem1, osem0, osem1):
        wid = lax.axis_index("s") * 2 + lax.axis_index("c")
        base_pt = wid * ppw
        rows_b = (rows_v0, rows_v1)
        mx_b = (mx_v0, mx_v1)
        s1_b = (s1_v0, s1_v1)
        s2_b = (s2_v0, s2_v1)
        gsem = (gsem0, gsem1)
        osem = (osem0, osem1)

        pltpu.sync_copy(idx_hbm.at[pl.ds(wid * nch, nch)], idx_v)

        def start_gather(ci, par):
            pltpu.make_async_copy(
                pt_hbm.at[idx_v.at[ci]], rows_b[par], gsem[par]).start()

        def wait_gather(ci, par):
            pltpu.make_async_copy(
                pt_hbm.at[idx_v.at[ci]], rows_b[par], gsem[par]).wait()

        def compute_chunk(ci, par):
            rows_v = rows_b[par]
            mx_v, s1_v, s2_v = mx_b[par], s1_b[par], s2_b[par]

            def point_body(p, _):
                base = p * KK
                himask = jnp.int32(-65536)

                def unpk(row, ws):
                    w = rows_v[row, pl.ds(ws, 16)]
                    a = lax.bitcast_convert_type(lax.shift_left(w, 16),
                                                 jnp.float32)
                    b = lax.bitcast_convert_type(lax.bitwise_and(w, himask),
                                                 jnp.float32)
                    return a, b

                def col_body(wb, _):
                    ws = wb * 16
                    nat = wb * 32
                    va, vb = unpk(base, ws)
                    ma, mb = va, vb
                    sa, sb = va, vb
                    qa, qb = va * va, vb * vb
                    for r in range(1, KK):
                        va, vb = unpk(base + r, ws)
                        ma = jnp.maximum(ma, va)
                        mb = jnp.maximum(mb, vb)
                        sa = sa + va
                        sb = sb + vb
                        qa = qa + va * va
                        qb = qb + vb * vb
                    mx_v[p, pl.ds(nat, 16)] = ma
                    mx_v[p, pl.ds(nat + 16, 16)] = mb
                    s1_v[p, pl.ds(nat, 16)] = sa
                    s1_v[p, pl.ds(nat + 16, 16)] = sb
                    s2_v[p, pl.ds(nat, 16)] = qa
                    s2_v[p, pl.ds(nat + 16, 16)] = qb
                    return 0

                lax.fori_loop(0, CO // 32, col_body, 0)
                return 0

            lax.fori_loop(0, CHP, point_body, 0)
            p0 = base_pt + ci * CHP
            pltpu.make_async_copy(mx_v, mx_hbm.at[pl.ds(p0, CHP)],
                                  osem[par]).start()
            pltpu.make_async_copy(s1_v, s1_hbm.at[pl.ds(p0, CHP)],
                                  osem[par]).start()
            pltpu.make_async_copy(s2_v, s2_hbm.at[pl.ds(p0, CHP)],
                                  osem[par]).start()

        def wait_out(par):
            pltpu.make_async_copy(mx_b[par], mx_hbm.at[pl.ds(0, CHP)],
                                  osem[par]).wait()
            pltpu.make_async_copy(s1_b[par], s1_hbm.at[pl.ds(0, CHP)],
                                  osem[par]).wait()
            pltpu.make_async_copy(s2_b[par], s2_hbm.at[pl.ds(0, CHP)],
                                  osem[par]).wait()

        start_gather(0, 0)

        def pair_body(pi, _):
            for q in range(2):
                ci = pi * 2 + q
                par = q

                @pl.when(ci + 1 < nch)
                def _qnext():
                    start_gather(ci + 1, 1 - par)

                wait_gather(ci, par)

                @pl.when(ci >= 2)
                def _drain():
                    wait_out(par)

                compute_chunk(ci, par)
            return 0

        lax.fori_loop(0, nch // 2, pair_body, 0)
        wait_out(0)
        wait_out(1)

    return body(pt_flat, idx2d)


_RED_T = 512


def _stats_body(nsteps, s1_ref, s2_ref, qt_ref, out_ref, acc_ref):
    step = pl.program_id(0)

    @pl.when(step == 0)
    def _init():
        acc_ref[...] = jnp.zeros_like(acc_ref)

    s1 = s1_ref[...]
    s2 = s2_ref[...]
    qt = qt_ref[0]
    sum_y = jnp.sum(s1 + KK * qt, axis=0, keepdims=True)
    sum_y2 = jnp.sum(s2 + 2.0 * s1 * qt + KK * qt * qt, axis=0,
                     keepdims=True)
    acc_ref[0:1, :] += sum_y
    acc_ref[1:2, :] += sum_y2

    @pl.when(step == nsteps - 1)
    def _fin():
        out_ref[...] = acc_ref[...]


def _stats(s1, s2, qt3, off, nb):
    nsteps = nb * NN // _RED_T
    soff = off * NN // _RED_T
    return pl.pallas_call(
        functools.partial(_stats_body, nsteps),
        grid=(nsteps,),
        in_specs=[
            pl.BlockSpec((_RED_T, CO), lambda i: (i, 0)),
            pl.BlockSpec((_RED_T, CO), lambda i: (i, 0)),
            pl.BlockSpec((1, _RED_T, CO), lambda i: (i + soff, 0, 0)),
        ],
        out_specs=pl.BlockSpec((2, CO), lambda i: (0, 0)),
        out_shape=jax.ShapeDtypeStruct((2, CO), jnp.float32),
        scratch_shapes=[pltpu.VMEM((2, CO), jnp.float32)],
    )(s1, s2, qt3)


def _fin_body(nsums, m_ref, qt_ref, g_ref, be_ref, w1_ref, b1_ref,
              w2_ref, b2_ref, *rest):
    sum_refs = rest[:nsums]
    out_ref = rest[nsums]
    acc = sum_refs[0][...]
    for r in sum_refs[1:]:
        acc = acc + r[...]
    mean = acc[0:1, :] / _MTOT
    var = acc[1:2, :] / _MTOT - mean * mean
    istd = lax.rsqrt(var + 1e-5)
    m = m_ref[0]
    qt = qt_ref[0]
    y = ((m + qt) - mean) * istd * g_ref[...] + be_ref[...]
    y = jnp.maximum(y, 0.0)
    w = jnp.sum(y, axis=0, keepdims=True) * (1.0 / NN)
    h = lax.dot_general(w, w1_ref[...], (((1,), (1,)), ((), ())),
                        preferred_element_type=jnp.float32)
    h = jnp.maximum(h + b1_ref[...], 0.0)
    s = lax.dot_general(h, w2_ref[...], (((1,), (1,)), ((), ())),
                        preferred_element_type=jnp.float32)
    s = jax.nn.sigmoid(s + b2_ref[...])
    out_ref[0] = jnp.transpose(y) * jnp.transpose(s)


def _finalize(mx, qt, sums, gamma2, beta2, W1, b12, W2, b22, off, nb):
    hid = W1.shape[0]
    nsums = len(sums)
    full = lambda b: (0, 0)
    return pl.pallas_call(
        functools.partial(_fin_body, nsums),
        grid=(nb,),
        in_specs=[
            pl.BlockSpec((1, NN, CO), lambda b: (b, 0, 0)),
            pl.BlockSpec((1, NN, CO), lambda b: (b + off, 0, 0)),
            pl.BlockSpec((1, CO), full),
            pl.BlockSpec((1, CO), full),
            pl.BlockSpec((hid, CO), full),
            pl.BlockSpec((1, hid), full),
            pl.BlockSpec((CO, hid), full),
            pl.BlockSpec((1, CO), full),
        ] + [pl.BlockSpec((2, CO), full) for _ in range(nsums)],
        out_specs=pl.BlockSpec((1, CO, NN), lambda b: (b, 0, 0)),
        out_shape=jax.ShapeDtypeStruct((nb, CO, NN), jnp.float32),
    )(mx, qt, gamma2, beta2, W1, b12, W2, b22, *sums)


def kernel(x, W_conv, gamma, beta, W1, b1, W2, b2):
    wa = W_conv[:, :CC]
    wd = W_conv[:, CC:] - wa
    wa_perm = wa[_PPERM]

    pt, qt, sq = _prep(x, wa_perm, wd)
    pt_flat = pt.reshape(BB * NN, CO // 2)
    qt3 = qt.reshape(BB * NN // _RED_T, _RED_T, CO)

    mxs, sums = [], []
    off = 0
    for nb in SPLITS:
        idx = _knn(x, sq, off, nb)
        mx, s1, s2 = _sc_gather_reduce(
            pt_flat, idx.reshape(nb * NN * KK // _RPC, _RPC), nb)
        sums.append(_stats(s1, s2, qt3, off, nb))
        mxs.append(mx)
        off += nb

    outs = []
    off = 0
    for mx, nb in zip(mxs, SPLITS):
        outs.append(_finalize(
            mx.reshape(nb, NN, CO), qt, sums,
            gamma.reshape(1, CO), beta.reshape(1, CO),
            W1, b1.reshape(1, -1), W2, b2.reshape(1, CO), off, nb))
        off += nb
    return jnp.concatenate(outs, axis=0)


# speedup vs baseline: 1.1794x; 1.0245x over previous
import functools

import jax
import jax.numpy as jnp
import numpy as np
from jax import lax
from jax.experimental import pallas as pl
from jax.experimental.pallas import tpu as pltpu
from jax.experimental.pallas import tpu_sc as plsc

BB, CC, NN, KK = 8, 256, 2048, 16
CO = 256
TN = 256
NW = 32
CHP = 8
_RPC = CHP * KK
SPLITS = (1,) * 8
_MTOT = float(BB * NN * KK)

_j = np.arange(128)
_PPERM = np.concatenate([32 * (_j // 16) + (_j % 16),
                         32 * (_j // 16) + 16 + (_j % 16)])


def _prep_body(x_ref, wa_ref, wd_ref, pt_ref, qt_ref, sq_ref):
    xb = x_ref[0]
    ptv = lax.dot_general(xb, wa_ref[...], (((0,), (1,)), ((), ())),
                          preferred_element_type=jnp.float32)

    def rnd(v):
        b = lax.bitcast_convert_type(v, jnp.int32)
        b = b + jnp.int32(0x7FFF) + lax.bitwise_and(
            lax.shift_right_logical(b, 16), jnp.int32(1))
        return lax.shift_right_arithmetic(b, 16)

    lo = lax.bitwise_and(rnd(ptv[:, :CO // 2]), jnp.int32(0xFFFF))
    hi = lax.shift_left(rnd(ptv[:, CO // 2:]), 16)
    pt_ref[0] = lax.bitwise_or(lo, hi)
    qt_ref[0] = lax.dot_general(xb, wd_ref[...], (((0,), (1,)), ((), ())),
                                preferred_element_type=jnp.float32)
    sq_ref[0, 0] = jnp.sum(xb * xb, axis=0)


def _prep(x, wa, wd):
    return pl.pallas_call(
        _prep_body,
        grid=(BB,),
        in_specs=[
            pl.BlockSpec((1, CC, NN), lambda b: (b, 0, 0)),
            pl.BlockSpec((CO, CC), lambda b: (0, 0)),
            pl.BlockSpec((CO, CC), lambda b: (0, 0)),
        ],
        out_specs=[
            pl.BlockSpec((1, NN, CO // 2), lambda b: (b, 0, 0)),
            pl.BlockSpec((1, NN, CO), lambda b: (b, 0, 0)),
            pl.BlockSpec((1, 1, NN), lambda b: (b, 0, 0)),
        ],
        out_shape=[
            jax.ShapeDtypeStruct((BB, NN, CO // 2), jnp.int32),
            jax.ShapeDtypeStruct((BB, NN, CO), jnp.float32),
            jax.ShapeDtypeStruct((BB, 1, NN), jnp.float32),
        ],
    )(x, wa, wd)


def _knn_body(off, xrow_ref, xall_ref, sq_ref, sqrow_ref, idx_ref):
    b = pl.program_id(0)
    xr = xrow_ref[0]
    xa = xall_ref[0]
    inner = lax.dot_general(xr, xa, (((0,), (0,)), ((), ())),
                            preferred_element_type=jnp.float32)
    sq = sq_ref[0]
    sqr = sqrow_ref[0]
    d = jnp.transpose(sqr) + sq - 2.0 * inner
    col_ids = lax.broadcasted_iota(jnp.int32, (TN, NN), 1)
    cols = []
    for _ in range(KK):
        am = jnp.argmin(d, axis=1).astype(jnp.int32).reshape(TN, 1)
        cols.append(am)
        d = jnp.where(col_ids == am, jnp.inf, d)
    idx_ref[0] = jnp.concatenate(cols, axis=1) + (b + off) * NN


def _knn(x, sq, off, nb):
    return pl.pallas_call(
        functools.partial(_knn_body, off),
        grid=(nb, NN // TN),
        in_specs=[
            pl.BlockSpec((1, CC, TN), lambda b, j: (b + off, 0, j)),
            pl.BlockSpec((1, CC, NN), lambda b, j: (b + off, 0, 0)),
            pl.BlockSpec((1, 1, NN), lambda b, j: (b + off, 0, 0)),
            pl.BlockSpec((1, 1, TN), lambda b, j: (b + off, 0, j)),
        ],
        out_specs=pl.BlockSpec((1, TN, KK), lambda b, j: (b, j, 0)),
        out_shape=jax.ShapeDtypeStruct((nb, NN, KK), jnp.int32),
    )(x, x, sq, sq)


def _sc_gather_reduce(pt_flat, idx2d, nb):
    ppw = nb * NN // NW
    nch = ppw // CHP
    mesh = plsc.VectorSubcoreMesh(core_axis_name="c", subcore_axis_name="s")

    @functools.partial(
        pl.kernel,
        mesh=mesh,
        out_type=(
            jax.ShapeDtypeStruct((nb * NN, CO), jnp.float32),
            jax.ShapeDtypeStruct((nb * NN, CO), jnp.float32),
            jax.ShapeDtypeStruct((nb * NN, CO), jnp.float32),
        ),
        name="sc_gather_reduce",
        scratch_types=[
            pltpu.VMEM((nch, _RPC), jnp.int32),
            pltpu.VMEM((_RPC, CO // 2), jnp.int32),
            pltpu.VMEM((_RPC, CO // 2), jnp.int32),
            pltpu.VMEM((CHP, CO), jnp.float32),
            pltpu.VMEM((CHP, CO), jnp.float32),
            pltpu.VMEM((CHP, CO), jnp.float32),
            pltpu.VMEM((CHP, CO), jnp.float32),
            pltpu.VMEM((CHP, CO), jnp.float32),
            pltpu.VMEM((CHP, CO), jnp.float32),
            pltpu.SemaphoreType.DMA,
            pltpu.SemaphoreType.DMA,
            pltpu.SemaphoreType.DMA,
            pltpu.SemaphoreType.DMA,
        ],
    )
    def body(pt_hbm, idx_hbm, mx_hbm, s1_hbm, s2_hbm,
             idx_v, rows_v0, rows_v1, mx_v0, mx_v1, s1_v0, s1_v1,
             s2_v0, s2_v1, gsem0, gsem1, osem0, osem1):
        wid = lax.axis_index("s") * 2 + lax.axis_index("c")
        base_pt = wid * ppw
        rows_b = (rows_v0, rows_v1)
        mx_b = (mx_v0, mx_v1)
        s1_b = (s1_v0, s1_v1)
        s2_b = (s2_v0, s2_v1)
        gsem = (gsem0, gsem1)
        osem = (osem0, osem1)

        pltpu.sync_copy(idx_hbm.at[pl.ds(wid * nch, nch)], idx_v)

        def start_gather(ci, par):
            pltpu.make_async_copy(
                pt_hbm.at[idx_v.at[ci]], rows_b[par], gsem[par]).start()

        def wait_gather(ci, par):
            pltpu.make_async_copy(
                pt_hbm.at[idx_v.at[ci]], rows_b[par], gsem[par]).wait()

        def compute_chunk(ci, par):
            rows_v = rows_b[par]
            mx_v, s1_v, s2_v = mx_b[par], s1_b[par], s2_b[par]

            def point_body(p, _):
                base = p * KK
                himask = jnp.int32(-65536)

                def unpk(row, ws):
                    w = rows_v[row, pl.ds(ws, 16)]
                    a = lax.bitcast_convert_type(lax.shift_left(w, 16),
                                                 jnp.float32)
                    b = lax.bitcast_convert_type(lax.bitwise_and(w, himask),
                                                 jnp.float32)
                    return a, b

                def col_body(wb, _):
                    ws = wb * 16
                    nat = wb * 32
                    va, vb = unpk(base, ws)
                    ma, mb = va, vb
                    sa, sb = va, vb
                    qa, qb = va * va, vb * vb
                    for r in range(1, KK):
                        va, vb = unpk(base + r, ws)
                        ma = jnp.maximum(ma, va)
                        mb = jnp.maximum(mb, vb)
                        sa = sa + va
                        sb = sb + vb
                        qa = qa + va * va
                        qb = qb + vb * vb
                    mx_v[p, pl.ds(nat, 16)] = ma
                    mx_v[p, pl.ds(nat + 16, 16)] = mb
                    s1_v[p, pl.ds(nat, 16)] = sa
                    s1_v[p, pl.ds(nat + 16, 16)] = sb
                    s2_v[p, pl.ds(nat, 16)] = qa
                    s2_v[p, pl.ds(nat + 16, 16)] = qb
                    return 0

                lax.fori_loop(0, CO // 32, col_body, 0)
                return 0

            lax.fori_loop(0, CHP, point_body, 0)
            p0 = base_pt + ci * CHP
            pltpu.make_async_copy(mx_v, mx_hbm.at[pl.ds(p0, CHP)],
                                  osem[par]).start()
            pltpu.make_async_copy(s1_v, s1_hbm.at[pl.ds(p0, CHP)],
                                  osem[par]).start()
            pltpu.make_async_copy(s2_v, s2_hbm.at[pl.ds(p0, CHP)],
                                  osem[par]).start()

        def wait_out(par):
            pltpu.make_async_copy(mx_b[par], mx_hbm.at[pl.ds(0, CHP)],
                                  osem[par]).wait()
            pltpu.make_async_copy(s1_b[par], s1_hbm.at[pl.ds(0, CHP)],
                                  osem[par]).wait()
            pltpu.make_async_copy(s2_b[par], s2_hbm.at[pl.ds(0, CHP)],
                                  osem[par]).wait()

        start_gather(0, 0)

        def pair_body(pi, _):
            for q in range(2):
                ci = pi * 2 + q
                par = q

                @pl.when(ci + 1 < nch)
                def _qnext():
                    start_gather(ci + 1, 1 - par)

                wait_gather(ci, par)

                @pl.when(ci >= 2)
                def _drain():
                    wait_out(par)

                compute_chunk(ci, par)
            return 0

        lax.fori_loop(0, nch // 2, pair_body, 0)
        wait_out(0)
        wait_out(1)

    return body(pt_flat, idx2d)


_RED_T = 512


def _stats_body(nsteps, s1_ref, s2_ref, qt_ref, out_ref, acc_ref):
    step = pl.program_id(0)

    @pl.when(step == 0)
    def _init():
        acc_ref[...] = jnp.zeros_like(acc_ref)

    s1 = s1_ref[...]
    s2 = s2_ref[...]
    qt = qt_ref[0]
    sum_y = jnp.sum(s1 + KK * qt, axis=0, keepdims=True)
    sum_y2 = jnp.sum(s2 + 2.0 * s1 * qt + KK * qt * qt, axis=0,
                     keepdims=True)
    acc_ref[0:1, :] += sum_y
    acc_ref[1:2, :] += sum_y2

    @pl.when(step == nsteps - 1)
    def _fin():
        out_ref[...] = acc_ref[...]


def _stats(s1, s2, qt3, off, nb):
    nsteps = nb * NN // _RED_T
    soff = off * NN // _RED_T
    return pl.pallas_call(
        functools.partial(_stats_body, nsteps),
        grid=(nsteps,),
        in_specs=[
            pl.BlockSpec((_RED_T, CO), lambda i: (i, 0)),
            pl.BlockSpec((_RED_T, CO), lambda i: (i, 0)),
            pl.BlockSpec((1, _RED_T, CO), lambda i: (i + soff, 0, 0)),
        ],
        out_specs=pl.BlockSpec((2, CO), lambda i: (0, 0)),
        out_shape=jax.ShapeDtypeStruct((2, CO), jnp.float32),
        scratch_shapes=[pltpu.VMEM((2, CO), jnp.float32)],
    )(s1, s2, qt3)


def _fin_body(nsums, m_ref, qt_ref, g_ref, be_ref, w1_ref, b1_ref,
              w2_ref, b2_ref, *rest):
    sum_refs = rest[:nsums]
    out_ref = rest[nsums]
    acc = sum_refs[0][...]
    for r in sum_refs[1:]:
        acc = acc + r[...]
    mean = acc[0:1, :] / _MTOT
    var = acc[1:2, :] / _MTOT - mean * mean
    istd = lax.rsqrt(var + 1e-5)
    m = m_ref[0]
    qt = qt_ref[0]
    y = ((m + qt) - mean) * istd * g_ref[...] + be_ref[...]
    y = jnp.maximum(y, 0.0)
    w = jnp.sum(y, axis=0, keepdims=True) * (1.0 / NN)
    h = lax.dot_general(w, w1_ref[...], (((1,), (1,)), ((), ())),
                        preferred_element_type=jnp.float32)
    h = jnp.maximum(h + b1_ref[...], 0.0)
    s = lax.dot_general(h, w2_ref[...], (((1,), (1,)), ((), ())),
                        preferred_element_type=jnp.float32)
    s = jax.nn.sigmoid(s + b2_ref[...])
    out_ref[0] = jnp.transpose(y) * jnp.transpose(s)


def _finalize(mx, qt, sums, gamma2, beta2, W1, b12, W2, b22, off, nb):
    hid = W1.shape[0]
    nsums = len(sums)
    full = lambda b: (0, 0)
    return pl.pallas_call(
        functools.partial(_fin_body, nsums),
        grid=(nb,),
        in_specs=[
            pl.BlockSpec((1, NN, CO), lambda b: (b, 0, 0)),
            pl.BlockSpec((1, NN, CO), lambda b: (b + off, 0, 0)),
            pl.BlockSpec((1, CO), full),
            pl.BlockSpec((1, CO), full),
            pl.BlockSpec((hid, CO), full),
            pl.BlockSpec((1, hid), full),
            pl.BlockSpec((CO, hid), full),
            pl.BlockSpec((1, CO), full),
        ] + [pl.BlockSpec((2, CO), full) for _ in range(nsums)],
        out_specs=pl.BlockSpec((1, CO, NN), lambda b: (b, 0, 0)),
        out_shape=jax.ShapeDtypeStruct((nb, CO, NN), jnp.float32),
    )(mx, qt, gamma2, beta2, W1, b12, W2, b22, *sums)


def kernel(x, W_conv, gamma, beta, W1, b1, W2, b2):
    wa = W_conv[:, :CC]
    wd = W_conv[:, CC:] - wa
    wa_perm = wa[_PPERM]

    pt, qt, sq = _prep(x, wa_perm, wd)
    pt_flat = pt.reshape(BB * NN, CO // 2)
    qt3 = qt.reshape(BB * NN // _RED_T, _RED_T, CO)

    mxs, sums = [], []
    off = 0
    for nb in SPLITS:
        idx = _knn(x, sq, off, nb)
        mx, s1, s2 = _sc_gather_reduce(
            pt_flat, idx.reshape(nb * NN * KK // _RPC, _RPC), nb)
        sums.append(_stats(s1, s2, qt3, off, nb))
        mxs.append(mx)
        off += nb

    outs = []
    off = 0
    for mx, nb in zip(mxs, SPLITS):
        outs.append(_finalize(
            mx.reshape(nb, NN, CO), qt, sums,
            gamma.reshape(1, CO), beta.reshape(1, CO),
            W1, b1.reshape(1, -1), W2, b2.reshape(1, CO), off, nb))
        off += nb
    return jnp.concatenate(outs, axis=0)


# speedup vs baseline: 1.2291x; 1.0421x over previous
import functools

import jax
import jax.numpy as jnp
import numpy as np
from jax import lax
from jax.experimental import pallas as pl
from jax.experimental.pallas import tpu as pltpu
from jax.experimental.pallas import tpu_sc as plsc

BB, CC, NN, KK = 8, 256, 2048, 16
CO = 256
TN = 256
NW = 32
CHP = 8
_RPC = CHP * KK
SPLITS = (1,) * 8
_MTOT = float(BB * NN * KK)

_j = np.arange(128)
_PPERM = np.concatenate([32 * (_j // 16) + (_j % 16),
                         32 * (_j // 16) + 16 + (_j % 16)])


def _prep_body(x_ref, wa_ref, wd_ref, pt_ref, qt_ref, sq_ref):
    xb = x_ref[0]
    ptv = lax.dot_general(xb, wa_ref[...], (((0,), (1,)), ((), ())),
                          preferred_element_type=jnp.float32)

    def rnd(v):
        b = lax.bitcast_convert_type(v, jnp.int32)
        b = b + jnp.int32(0x7FFF) + lax.bitwise_and(
            lax.shift_right_logical(b, 16), jnp.int32(1))
        return lax.shift_right_arithmetic(b, 16)

    lo = lax.bitwise_and(rnd(ptv[:, :CO // 2]), jnp.int32(0xFFFF))
    hi = lax.shift_left(rnd(ptv[:, CO // 2:]), 16)
    pt_ref[0] = lax.bitwise_or(lo, hi)
    qt_ref[0] = lax.dot_general(xb, wd_ref[...], (((0,), (1,)), ((), ())),
                                preferred_element_type=jnp.float32)
    sq_ref[0, 0] = jnp.sum(xb * xb, axis=0)


def _prep(x, wa, wd):
    return pl.pallas_call(
        _prep_body,
        grid=(BB,),
        in_specs=[
            pl.BlockSpec((1, CC, NN), lambda b: (b, 0, 0)),
            pl.BlockSpec((CO, CC), lambda b: (0, 0)),
            pl.BlockSpec((CO, CC), lambda b: (0, 0)),
        ],
        out_specs=[
            pl.BlockSpec((1, NN, CO // 2), lambda b: (b, 0, 0)),
            pl.BlockSpec((1, NN, CO), lambda b: (b, 0, 0)),
            pl.BlockSpec((1, 1, NN), lambda b: (b, 0, 0)),
        ],
        out_shape=[
            jax.ShapeDtypeStruct((BB, NN, CO // 2), jnp.int32),
            jax.ShapeDtypeStruct((BB, NN, CO), jnp.float32),
            jax.ShapeDtypeStruct((BB, 1, NN), jnp.float32),
        ],
    )(x, wa, wd)


def _knn_body(off, xrow_ref, xall_ref, sq_ref, sqrow_ref, idx_ref):
    b = pl.program_id(0)
    xr = xrow_ref[0]
    xa = xall_ref[0]
    inner = lax.dot_general(xr, xa, (((0,), (0,)), ((), ())),
                            preferred_element_type=jnp.float32)
    sq = sq_ref[0]
    sqr = sqrow_ref[0]
    d = jnp.transpose(sqr) + sq - 2.0 * inner
    col_ids = lax.broadcasted_iota(jnp.int32, (TN, NN), 1)
    cols = []
    for _ in range(KK):
        am = jnp.argmin(d, axis=1).astype(jnp.int32).reshape(TN, 1)
        cols.append(am)
        d = jnp.where(col_ids == am, jnp.inf, d)
    idx_ref[0] = jnp.concatenate(cols, axis=1) + (b + off) * NN


def _knn(x, sq, off, nb):
    return pl.pallas_call(
        functools.partial(_knn_body, off),
        grid=(nb, NN // TN),
        in_specs=[
            pl.BlockSpec((1, CC, TN), lambda b, j: (b + off, 0, j)),
            pl.BlockSpec((1, CC, NN), lambda b, j: (b + off, 0, 0)),
            pl.BlockSpec((1, 1, NN), lambda b, j: (b + off, 0, 0)),
            pl.BlockSpec((1, 1, TN), lambda b, j: (b + off, 0, j)),
        ],
        out_specs=pl.BlockSpec((1, TN, KK), lambda b, j: (b, j, 0)),
        out_shape=jax.ShapeDtypeStruct((nb, NN, KK), jnp.int32),
    )(x, x, sq, sq)


def _sc_gather_reduce(pt_flat, idx2d, qt_flat, qoff, nb):
    ppw = nb * NN // NW
    nch = ppw // CHP
    mesh = plsc.VectorSubcoreMesh(core_axis_name="c", subcore_axis_name="s")

    @functools.partial(
        pl.kernel,
        mesh=mesh,
        out_type=(
            jax.ShapeDtypeStruct((nb * NN, CO), jnp.float32),
            jax.ShapeDtypeStruct((NW, 2, CO), jnp.float32),
        ),
        name="sc_gather_reduce",
        scratch_types=[
            pltpu.VMEM((nch, _RPC), jnp.int32),
            pltpu.VMEM((ppw, CO), jnp.float32),
            pltpu.VMEM((2, CO), jnp.float32),
            pltpu.VMEM((_RPC, CO // 2), jnp.int32),
            pltpu.VMEM((_RPC, CO // 2), jnp.int32),
            pltpu.VMEM((CHP, CO), jnp.float32),
            pltpu.VMEM((CHP, CO), jnp.float32),
            pltpu.SemaphoreType.DMA,
            pltpu.SemaphoreType.DMA,
            pltpu.SemaphoreType.DMA,
            pltpu.SemaphoreType.DMA,
        ],
    )
    def body(pt_hbm, idx_hbm, qt_hbm, mx_hbm, ps_hbm,
             idx_v, qt_v, acc_v, rows_v0, rows_v1, mx_v0, mx_v1,
             gsem0, gsem1, osem0, osem1):
        wid = lax.axis_index("s") * 2 + lax.axis_index("c")
        base_pt = wid * ppw
        rows_b = (rows_v0, rows_v1)
        mx_b = (mx_v0, mx_v1)
        gsem = (gsem0, gsem1)
        osem = (osem0, osem1)

        pltpu.sync_copy(idx_hbm.at[pl.ds(wid * nch, nch)], idx_v)
        pltpu.sync_copy(qt_hbm.at[pl.ds(qoff + base_pt, ppw)], qt_v)
        zero16 = jnp.zeros((16,), jnp.float32)

        def zinit(c, _):
            acc_v[0, pl.ds(c * 16, 16)] = zero16
            acc_v[1, pl.ds(c * 16, 16)] = zero16
            return 0

        lax.fori_loop(0, CO // 16, zinit, 0)

        def start_gather(ci, par):
            pltpu.make_async_copy(
                pt_hbm.at[idx_v.at[ci]], rows_b[par], gsem[par]).start()

        def wait_gather(ci, par):
            pltpu.make_async_copy(
                pt_hbm.at[idx_v.at[ci]], rows_b[par], gsem[par]).wait()

        def compute_chunk(ci, par):
            rows_v = rows_b[par]
            mx_v = mx_b[par]

            def point_body(p, _):
                base = p * KK
                pg = ci * CHP + p
                himask = jnp.int32(-65536)

                def unpk(row, ws):
                    w = rows_v[row, pl.ds(ws, 16)]
                    a = lax.bitcast_convert_type(lax.shift_left(w, 16),
                                                 jnp.float32)
                    b = lax.bitcast_convert_type(lax.bitwise_and(w, himask),
                                                 jnp.float32)
                    return a, b

                def col_body(wb, _):
                    ws = wb * 16
                    nat = wb * 32
                    va, vb = unpk(base, ws)
                    ma, mb = va, vb
                    sa, sb = va, vb
                    qa, qb = va * va, vb * vb
                    for r in range(1, KK):
                        va, vb = unpk(base + r, ws)
                        ma = jnp.maximum(ma, va)
                        mb = jnp.maximum(mb, vb)
                        sa = sa + va
                        sb = sb + vb
                        qa = qa + va * va
                        qb = qb + vb * vb
                    mx_v[p, pl.ds(nat, 16)] = ma
                    mx_v[p, pl.ds(nat + 16, 16)] = mb
                    qva = qt_v[pg, pl.ds(nat, 16)]
                    qvb = qt_v[pg, pl.ds(nat + 16, 16)]
                    kf = jnp.float32(KK)
                    acc_v[0, pl.ds(nat, 16)] += sa + kf * qva
                    acc_v[0, pl.ds(nat + 16, 16)] += sb + kf * qvb
                    acc_v[1, pl.ds(nat, 16)] += (qa + 2.0 * sa * qva
                                                 + kf * qva * qva)
                    acc_v[1, pl.ds(nat + 16, 16)] += (qb + 2.0 * sb * qvb
                                                      + kf * qvb * qvb)
                    return 0

                lax.fori_loop(0, CO // 32, col_body, 0)
                return 0

            lax.fori_loop(0, CHP, point_body, 0)
            p0 = base_pt + ci * CHP
            pltpu.make_async_copy(mx_v, mx_hbm.at[pl.ds(p0, CHP)],
                                  osem[par]).start()

        def wait_out(par):
            pltpu.make_async_copy(mx_b[par], mx_hbm.at[pl.ds(0, CHP)],
                                  osem[par]).wait()

        start_gather(0, 0)

        def pair_body(pi, _):
            for q in range(2):
                ci = pi * 2 + q
                par = q

                @pl.when(ci + 1 < nch)
                def _qnext():
                    start_gather(ci + 1, 1 - par)

                wait_gather(ci, par)

                @pl.when(ci >= 2)
                def _drain():
                    wait_out(par)

                compute_chunk(ci, par)
            return 0

        lax.fori_loop(0, nch // 2, pair_body, 0)
        wait_out(0)
        wait_out(1)
        pltpu.sync_copy(acc_v, ps_hbm.at[wid])

    return body(pt_flat, idx2d, qt_flat)


def _fin_body(nsums, m_ref, qt_ref, g_ref, be_ref, w1_ref, b1_ref,
              w2_ref, b2_ref, *rest):
    sum_refs = rest[:nsums]
    out_ref = rest[nsums]
    acc = jnp.sum(sum_refs[0][...], axis=0)
    for r in sum_refs[1:]:
        acc = acc + jnp.sum(r[...], axis=0)
    mean = acc[0:1, :] / _MTOT
    var = acc[1:2, :] / _MTOT - mean * mean
    istd = lax.rsqrt(var + 1e-5)
    m = m_ref[0]
    qt = qt_ref[0]
    y = ((m + qt) - mean) * istd * g_ref[...] + be_ref[...]
    y = jnp.maximum(y, 0.0)
    w = jnp.sum(y, axis=0, keepdims=True) * (1.0 / NN)
    h = lax.dot_general(w, w1_ref[...], (((1,), (1,)), ((), ())),
                        preferred_element_type=jnp.float32)
    h = jnp.maximum(h + b1_ref[...], 0.0)
    s = lax.dot_general(h, w2_ref[...], (((1,), (1,)), ((), ())),
                        preferred_element_type=jnp.float32)
    s = jax.nn.sigmoid(s + b2_ref[...])
    out_ref[0] = jnp.transpose(y) * jnp.transpose(s)


def _finalize(mx, qt, sums, gamma2, beta2, W1, b12, W2, b22, off, nb):
    hid = W1.shape[0]
    nsums = len(sums)
    full = lambda b: (0, 0)
    return pl.pallas_call(
        functools.partial(_fin_body, nsums),
        grid=(nb,),
        in_specs=[
            pl.BlockSpec((1, NN, CO), lambda b: (b, 0, 0)),
            pl.BlockSpec((1, NN, CO), lambda b: (b + off, 0, 0)),
            pl.BlockSpec((1, CO), full),
            pl.BlockSpec((1, CO), full),
            pl.BlockSpec((hid, CO), full),
            pl.BlockSpec((1, hid), full),
            pl.BlockSpec((CO, hid), full),
            pl.BlockSpec((1, CO), full),
        ] + [pl.BlockSpec((NW, 2, CO), lambda b: (0, 0, 0))
             for _ in range(nsums)],
        out_specs=pl.BlockSpec((1, CO, NN), lambda b: (b, 0, 0)),
        out_shape=jax.ShapeDtypeStruct((nb, CO, NN), jnp.float32),
    )(mx, qt, gamma2, beta2, W1, b12, W2, b22, *sums)


def kernel(x, W_conv, gamma, beta, W1, b1, W2, b2):
    wa = W_conv[:, :CC]
    wd = W_conv[:, CC:] - wa
    wa_perm = wa[_PPERM]

    pt, qt, sq = _prep(x, wa_perm, wd)
    pt_flat = pt.reshape(BB * NN, CO // 2)
    qt_flat = qt.reshape(BB * NN, CO)

    mxs, sums = [], []
    off = 0
    for nb in SPLITS:
        idx = _knn(x, sq, off, nb)
        mx, ps = _sc_gather_reduce(
            pt_flat, idx.reshape(nb * NN * KK // _RPC, _RPC),
            qt_flat, off * NN, nb)
        sums.append(ps)
        mxs.append(mx)
        off += nb

    outs = []
    off = 0
    for mx, nb in zip(mxs, SPLITS):
        outs.append(_finalize(
            mx.reshape(nb, NN, CO), qt, sums,
            gamma.reshape(1, CO), beta.reshape(1, CO),
            W1, b1.reshape(1, -1), W2, b2.reshape(1, CO), off, nb))
        off += nb
    return jnp.concatenate(outs, axis=0)
